# baseline (speedup 1.0000x reference)
import functools

import jax
import jax.numpy as jnp
from jax import lax
from jax.experimental import pallas as pl
from jax.experimental.pallas import tpu as pltpu
from jax.experimental.pallas import tpu_sc as plsc

_NBUF = 4
_CHUNK = 256


def _gather_kernel(n_rows, hidden, num_cores, num_subcores):
    num_workers = num_cores * num_subcores
    per_w = n_rows // num_workers
    chunk = _CHUNK
    n_chunks = per_w // chunk
    n_iters = n_chunks // _NBUF

    mesh = plsc.VectorSubcoreMesh(core_axis_name="c", subcore_axis_name="s")

    @functools.partial(
        pl.kernel,
        mesh=mesh,
        compiler_params=pltpu.CompilerParams(use_tc_tiling_on_sc=False),
        out_type=jax.ShapeDtypeStruct((n_rows, hidden), jnp.int32),
        scratch_types=[
            pltpu.VMEM((per_w,), jnp.int32),
            pltpu.VMEM((_NBUF, chunk, hidden), jnp.int32),
            [pltpu.SemaphoreType.DMA] * _NBUF,
            [pltpu.SemaphoreType.DMA] * _NBUF,
        ],
    )
    def body(idx_hbm, table_hbm, out_hbm, idx_v, bufs, gsems, ssems):
        wid = lax.axis_index("s") * num_cores + lax.axis_index("c")
        base = wid * per_w
        pltpu.sync_copy(idx_hbm.at[pl.ds(base, per_w)], idx_v)

        def start_gather(j, b):
            pltpu.async_copy(
                table_hbm.at[idx_v.at[pl.ds(j * chunk, chunk)]],
                bufs.at[b], gsems[b])

        for b in range(_NBUF):
            start_gather(b, b)

        def outer(i, carry):
            for b in range(_NBUF):
                j = i * _NBUF + b
                pltpu.make_async_copy(
                    table_hbm.at[idx_v.at[pl.ds(0, chunk)]],
                    bufs.at[b], gsems[b]).wait()
                pltpu.make_async_copy(
                    bufs.at[b],
                    out_hbm.at[pl.ds(base + j * chunk, chunk)],
                    ssems[b]).start()

            @pl.when(i < n_iters - 1)
            def _():
                for b in range(_NBUF):
                    pltpu.make_async_copy(
                        bufs.at[b], out_hbm.at[pl.ds(base, chunk)],
                        ssems[b]).wait()
                    start_gather((i + 1) * _NBUF + b, b)

            return carry

        lax.fori_loop(0, n_iters, outer, 0)
        for b in range(_NBUF):
            pltpu.make_async_copy(
                bufs.at[b], out_hbm.at[pl.ds(base, chunk)], ssems[b]).wait()

    return body


def kernel(x, table):
    b, l = x.shape
    vocab, hidden = table.shape
    n_rows = b * l
    info = plsc.get_sparse_core_info()
    idx = x.reshape(n_rows).astype(jnp.int32)
    t16 = jax.lax.bitcast_convert_type(table, jnp.uint16).astype(jnp.uint32)
    table_i32 = (t16[:, 0::2] | (t16[:, 1::2] << 16)).astype(jnp.int32)
    fn = _gather_kernel(n_rows, hidden // 2,
                        info.num_cores, info.num_subcores)
    out = fn(idx, table_i32).astype(jnp.uint32)
    lo = jax.lax.bitcast_convert_type(
        (out & 0xFFFF).astype(jnp.uint16), jnp.bfloat16)
    hi = jax.lax.bitcast_convert_type(
        (out >> 16).astype(jnp.uint16), jnp.bfloat16)
    res = jnp.stack([lo, hi], axis=-1).reshape(b, l, hidden)
    return res


# speedup vs baseline: 3.9029x; 3.9029x over previous
import functools

import jax
import jax.numpy as jnp
from jax import lax
from jax.experimental import pallas as pl
from jax.experimental.pallas import tpu as pltpu
from jax.experimental.pallas import tpu_sc as plsc

_NBUF = 2
_CHUNK = 256


def _gather_kernel(n_rows, hidden, num_cores, num_subcores):
    num_workers = num_cores * num_subcores
    per_w = n_rows // num_workers
    chunk = _CHUNK
    n_chunks = per_w // chunk
    n_iters = n_chunks // _NBUF

    mesh = plsc.VectorSubcoreMesh(core_axis_name="c", subcore_axis_name="s")

    @functools.partial(
        pl.kernel,
        mesh=mesh,
        compiler_params=pltpu.CompilerParams(
            use_tc_tiling_on_sc=False, needs_layout_passes=False),
        out_type=jax.ShapeDtypeStruct((n_rows, 2 * hidden), jnp.bfloat16),
        scratch_types=[
            pltpu.VMEM((per_w,), jnp.int32),
            pltpu.VMEM((_NBUF, chunk, hidden), jnp.int32),
            pltpu.VMEM((_NBUF, chunk, 2 * hidden), jnp.bfloat16),
            [pltpu.SemaphoreType.DMA] * _NBUF,
            [pltpu.SemaphoreType.DMA] * _NBUF,
        ],
    )
    def body(idx_hbm, table_hbm, out_bf16, idx_v, bufi, bufo, gsems, ssems):
        wid = lax.axis_index("s") * num_cores + lax.axis_index("c")
        base = wid * per_w
        pltpu.sync_copy(idx_hbm.at[pl.ds(base, per_w)], idx_v)

        def start_gather(j, b):
            pltpu.async_copy(
                table_hbm.at[idx_v.at[pl.ds(j * chunk, chunk)]],
                bufi.at[b], gsems[b])

        def wait_gather(b):
            pltpu.make_async_copy(
                table_hbm.at[idx_v.at[pl.ds(0, chunk)]],
                bufi.at[b], gsems[b]).wait()

        def wait_store(b):
            pltpu.make_async_copy(
                bufo.at[b], out_bf16.at[pl.ds(base, chunk)], ssems[b]).wait()

        def convert(b):
            src = bufi.at[b]
            dst = bufo.at[b]

            def rr(q, c):
                for k in range(hidden // 16):
                    v = src[q, pl.ds(16 * k, 16)]
                    dst[q, pl.ds(32 * k, 32)] = plsc.bitcast(
                        v, jnp.bfloat16)
                return c

            lax.fori_loop(0, chunk, rr, 0)

        for b in range(_NBUF):
            start_gather(b, b)

        def outer(i, carry):
            for b in range(_NBUF):
                j = i * _NBUF + b
                wait_gather(b)

                @pl.when(i > 0)
                def _():
                    wait_store(b)

                convert(b)

                @pl.when(i < n_iters - 1)
                def _():
                    start_gather((i + 1) * _NBUF + b, b)

                pltpu.make_async_copy(
                    bufo.at[b],
                    out_bf16.at[pl.ds(base + j * chunk, chunk)],
                    ssems[b]).start()

            return carry

        lax.fori_loop(0, n_iters, outer, 0)
        for b in range(_NBUF):
            wait_store(b)

    return body


def kernel(x, table):
    b, l = x.shape
    vocab, hidden = table.shape
    n_rows = b * l
    info = plsc.get_sparse_core_info()
    idx = x.reshape(n_rows).astype(jnp.int32)
    table_i32 = jax.lax.bitcast_convert_type(
        table.reshape(vocab, hidden // 2, 2), jnp.int32)
    fn = _gather_kernel(n_rows, hidden // 2,
                        info.num_cores, info.num_subcores)
    out = fn(idx, table_i32)
    return out.reshape(b, l, hidden)
